# Initial kernel scaffold; baseline (speedup 1.0000x reference)
import jax
import jax.numpy as jnp
from jax.experimental import pallas as pl


def kernel(x, memory, W, b):
    raise NotImplementedError("write your pallas kernel here")


# speedup vs baseline: 1.7110x; 1.7110x over previous
import functools

import jax
import jax.numpy as jnp
from jax import lax
from jax.experimental import pallas as pl
from jax.experimental.pallas import tpu as pltpu
from jax.experimental.pallas import tpu_sc as plsc

B = 64
M = 16384
CW = 64
R = 8
K = 8
IN = 512
C = K * R + 1
CP = 72
IFACE = CW * R + CW + 2

BLK = 2048
NB = M // BLK

_NEG = -3e38
_INTERPRET = False


def _iface_body(x_ref, w_ref, b_ref, q_ref, wv_ref, g_ref):
    xi = lax.dot_general(
        x_ref[...], w_ref[...], (((1,), (1,)), ((), ())),
        preferred_element_type=jnp.float32) + b_ref[...]
    q_ref[...] = xi[:, : R * CW]
    wv_ref[...] = jnp.tanh(xi[:, R * CW : R * CW + CW])
    ig = jax.nn.sigmoid(xi[:, R * CW + CW : R * CW + CW + 1])
    wg = jax.nn.sigmoid(xi[:, R * CW + CW + 1 : R * CW + CW + 2])
    g_ref[...] = jnp.concatenate(
        [ig, wg, jnp.zeros((B, 14), jnp.float32)], axis=1)


def _iface(x, W, b):
    return pl.pallas_call(
        _iface_body,
        out_shape=(
            jax.ShapeDtypeStruct((B, R * CW), jnp.float32),
            jax.ShapeDtypeStruct((B, CW), jnp.float32),
            jax.ShapeDtypeStruct((B, 16), jnp.float32),
        ),
        interpret=_INTERPRET,
    )(x, W, b.reshape(1, IFACE))


def _scan_body(q_ref, mem_ref, out_ref, idx_ref, rv_scr, ri_scr):
    m = pl.program_id(1)
    mem = mem_ref[0]
    out_ref[0] = mem
    s = lax.dot_general(
        q_ref[0], mem, (((1,), (1,)), ((), ())),
        preferred_element_type=jnp.float32)

    prev_v = jnp.where(m == 0, jnp.full((R, K), _NEG), rv_scr[...])
    prev_i = jnp.where(m == 0, jnp.zeros((R, K), jnp.int32), ri_scr[...])
    ioc = lax.broadcasted_iota(jnp.int32, (R, BLK), 1) + m * BLK
    cv = jnp.concatenate([prev_v, s], axis=1)
    ci = jnp.concatenate([prev_i, ioc], axis=1)
    lane = lax.broadcasted_iota(jnp.int32, cv.shape, 1)
    vs, iv = [], []
    for _ in range(K):
        mx = jnp.max(cv, axis=1, keepdims=True)
        pos = jnp.min(jnp.where(cv >= mx, lane, jnp.int32(1 << 30)),
                      axis=1, keepdims=True)
        sel = lane == pos
        vs.append(mx)
        iv.append(jnp.sum(jnp.where(sel, ci, 0), axis=1, keepdims=True))
        cv = jnp.where(sel, _NEG, cv)
    rv_scr[...] = jnp.concatenate(vs, axis=1)
    ri_scr[...] = jnp.concatenate(iv, axis=1)

    @pl.when(m == NB - 1)
    def _():
        idx_ref[0] = ri_scr[...]


def _scan(q3, memory):
    return pl.pallas_call(
        _scan_body,
        grid=(B, NB),
        in_specs=[
            pl.BlockSpec((1, R, CW), lambda b, m: (b, 0, 0)),
            pl.BlockSpec((1, BLK, CW), lambda b, m: (b, m, 0)),
        ],
        out_specs=[
            pl.BlockSpec((1, BLK, CW), lambda b, m: (b, m, 0)),
            pl.BlockSpec((1, R, K), lambda b, m: (b, 0, 0)),
        ],
        out_shape=(
            jax.ShapeDtypeStruct((B, M, CW), jnp.float32),
            jax.ShapeDtypeStruct((B, R, K), jnp.int32),
        ),
        scratch_shapes=[
            pltpu.VMEM((R, K), jnp.float32),
            pltpu.VMEM((R, K), jnp.int32),
        ],
        interpret=_INTERPRET,
    )(q3, memory)


_NC = 2
_NS = 16
_NW = _NC * _NS
_BPW = B // _NW

_SC_PARAMS = pltpu.CompilerParams(
    needs_layout_passes=False, use_tc_tiling_on_sc=False)


def _gather_body(pos_hbm, mem_hbm, vis_hbm, pos_v, vis_v, sem):
    wid = lax.axis_index("s") * _NC + lax.axis_index("c")
    for j in range(_BPW):
        b = wid * _BPW + j
        pltpu.sync_copy(pos_hbm.at[b], pos_v)
        pltpu.async_copy(mem_hbm.at[pos_v], vis_v, sem).wait()
        pltpu.sync_copy(vis_v, vis_hbm.at[b])


def _sc_gather(pos, mem2d):
    mesh = plsc.VectorSubcoreMesh(core_axis_name="c", subcore_axis_name="s")
    f = pl.kernel(
        _gather_body,
        out_type=jax.ShapeDtypeStruct((B, CP, CW), jnp.float32),
        mesh=mesh,
        scratch_types=[
            pltpu.VMEM((CP,), jnp.int32),
            pltpu.VMEM((CP, CW), jnp.float32),
            pltpu.SemaphoreType.DMA,
        ],
        compiler_params=_SC_PARAMS,
    )
    return f(pos, mem2d)


def _scatter_body(pos_hbm, vn_hbm, nm_ref, pos_v, vn_v, sem):
    wid = lax.axis_index("s") * _NC + lax.axis_index("c")
    for j in range(_BPW):
        b = wid * _BPW + j
        pltpu.sync_copy(pos_hbm.at[b], pos_v)
        pltpu.sync_copy(vn_hbm.at[b], vn_v)
        pltpu.async_copy(vn_v, nm_ref.at[pos_v], sem).wait()


def _sc_scatter(pos, vn, nm_ref):
    mesh = plsc.VectorSubcoreMesh(core_axis_name="c", subcore_axis_name="s")
    f = pl.kernel(
        _scatter_body,
        out_type=(),
        mesh=mesh,
        scratch_types=[
            pltpu.VMEM((CP,), jnp.int32),
            pltpu.VMEM((CP, CW), jnp.float32),
            pltpu.SemaphoreType.DMA,
        ],
        compiler_params=_SC_PARAMS,
    )
    return f(pos, vn, nm_ref)


def _attn_body(q_ref, vis_ref, wv_ref, g_ref, rv_ref, vn_ref):
    q = q_ref[0]
    vis = vis_ref[0]
    s = lax.dot_general(
        q, vis, (((1,), (1,)), ((), ())),
        preferred_element_type=jnp.float32)
    cols = lax.broadcasted_iota(jnp.int32, (R, CP), 1)
    s = jnp.where(cols < C, s, _NEG)
    mx = jnp.max(s, axis=1, keepdims=True)
    e = jnp.exp(s - mx)
    e = jnp.where(cols < C, e, 0.0)
    w = e / jnp.sum(e, axis=1, keepdims=True)
    rv_ref[0] = lax.dot_general(
        w, vis, (((1,), (0,)), ((), ())),
        preferred_element_type=jnp.float32)

    gv = g_ref[0, 0]
    ig = gv[0]
    wg = gv[1]
    ww = wg * (ig * jnp.mean(w, axis=0) + (1.0 - ig) / C)
    wvec = wv_ref[0, 0]
    vn = (vis * (1.0 - ww[:, None]) + ww[:, None] * wvec[None, :])
    row_lru = vn[C - 1 : C, :]
    rows = lax.broadcasted_iota(jnp.int32, (CP, CW), 0)
    vn_ref[0] = jnp.where(rows < C, vn, row_lru)


def _attn(q3, vis, wv, g):
    return pl.pallas_call(
        _attn_body,
        grid=(B,),
        in_specs=[
            pl.BlockSpec((1, R, CW), lambda b: (b, 0, 0)),
            pl.BlockSpec((1, CP, CW), lambda b: (b, 0, 0)),
            pl.BlockSpec((1, 1, CW), lambda b: (b, 0, 0)),
            pl.BlockSpec((1, 1, 16), lambda b: (b, 0, 0)),
        ],
        out_specs=[
            pl.BlockSpec((1, R, CW), lambda b: (b, 0, 0)),
            pl.BlockSpec((1, CP, CW), lambda b: (b, 0, 0)),
        ],
        out_shape=(
            jax.ShapeDtypeStruct((B, R, CW), jnp.float32),
            jax.ShapeDtypeStruct((B, CP, CW), jnp.float32),
        ),
        interpret=_INTERPRET,
    )(q3, vis, wv.reshape(B, 1, CW), g.reshape(B, 1, 16))


def kernel(x, memory, W, b):
    q, wv, g = _iface(x, W, b)
    q3 = q.reshape(B, R, CW)
    newmem, idx = _scan(q3, memory)

    idxf = idx.reshape(B, R * K)
    pos = jnp.concatenate(
        [idxf, jnp.zeros((B, CP - R * K), jnp.int32)], axis=1)
    pos = pos + (jnp.arange(B, dtype=jnp.int32) * M)[:, None]

    mem2d = memory.reshape(B * M, CW)
    vis = _sc_gather(pos, mem2d)
    rv, vn = _attn(q3, vis, wv, g)

    nm_ref = jax.new_ref(newmem.reshape(B * M, CW))
    _sc_scatter(pos, vn, nm_ref)
    new_memory = nm_ref[...].reshape(B, M, CW)
    return rv.reshape(B, R * CW), new_memory


# speedup vs baseline: 2.0114x; 1.1756x over previous
import functools

import jax
import jax.numpy as jnp
from jax import lax
from jax.experimental import pallas as pl
from jax.experimental.pallas import tpu as pltpu
from jax.experimental.pallas import tpu_sc as plsc

B = 64
M = 16384
CW = 64
R = 8
K = 8
IN = 512
C = K * R + 1
CP = 72
IFACE = CW * R + CW + 2

BLK = 2048
NB = M // BLK

_NEG = -3e38
_INTERPRET = False


def _iface_body(x_ref, w_ref, b_ref, q_ref, wv_ref, g_ref):
    xi = lax.dot_general(
        x_ref[...], w_ref[...], (((1,), (1,)), ((), ())),
        preferred_element_type=jnp.float32) + b_ref[...]
    q_ref[...] = xi[:, : R * CW]
    wv_ref[...] = jnp.tanh(xi[:, R * CW : R * CW + CW])
    ig = jax.nn.sigmoid(xi[:, R * CW + CW : R * CW + CW + 1])
    wg = jax.nn.sigmoid(xi[:, R * CW + CW + 1 : R * CW + CW + 2])
    g_ref[...] = jnp.concatenate(
        [ig, wg, jnp.zeros((B, 14), jnp.float32)], axis=1)


def _iface(x, W, b):
    return pl.pallas_call(
        _iface_body,
        out_shape=(
            jax.ShapeDtypeStruct((B, R * CW), jnp.float32),
            jax.ShapeDtypeStruct((B, CW), jnp.float32),
            jax.ShapeDtypeStruct((B, 16), jnp.float32),
        ),
        interpret=_INTERPRET,
    )(x, W, b.reshape(1, IFACE))


def _scan_body(q_ref, mem_ref, out_ref, idx_ref, s_scr):
    m = pl.program_id(1)
    mem = mem_ref[0]
    out_ref[0] = mem
    s_scr[m] = lax.dot_general(
        q_ref[0], mem, (((1,), (1,)), ((), ())),
        preferred_element_type=jnp.float32)

    @pl.when(m == NB - 1)
    def _():
        cv = jnp.concatenate([s_scr[i] for i in range(NB)], axis=1)
        lane = lax.broadcasted_iota(jnp.int32, (R, M), 1)
        ps = []
        for _ in range(K):
            mx = jnp.max(cv, axis=1, keepdims=True)
            pos = jnp.min(jnp.where(cv >= mx, lane, jnp.int32(1 << 30)),
                          axis=1, keepdims=True)
            ps.append(pos)
            cv = jnp.where(lane == pos, _NEG, cv)
        idx_ref[0] = jnp.concatenate(ps, axis=1)


def _scan(q3, memory):
    return pl.pallas_call(
        _scan_body,
        grid=(B, NB),
        in_specs=[
            pl.BlockSpec((1, R, CW), lambda b, m: (b, 0, 0)),
            pl.BlockSpec((1, BLK, CW), lambda b, m: (b, m, 0)),
        ],
        out_specs=[
            pl.BlockSpec((1, BLK, CW), lambda b, m: (b, m, 0)),
            pl.BlockSpec((1, R, K), lambda b, m: (b, 0, 0)),
        ],
        out_shape=(
            jax.ShapeDtypeStruct((B, M, CW), jnp.float32),
            jax.ShapeDtypeStruct((B, R, K), jnp.int32),
        ),
        scratch_shapes=[
            pltpu.VMEM((NB, R, BLK), jnp.float32),
        ],
        interpret=_INTERPRET,
    )(q3, memory)


_NC = 2
_NS = 16
_NW = _NC * _NS
_BPW = B // _NW

_SC_PARAMS = pltpu.CompilerParams(
    needs_layout_passes=False, use_tc_tiling_on_sc=False)


def _gather_body(pos_hbm, mem_hbm, vis_hbm, pos_v, vis_v, sem):
    wid = lax.axis_index("s") * _NC + lax.axis_index("c")
    for j in range(_BPW):
        b = wid * _BPW + j
        pltpu.sync_copy(pos_hbm.at[b], pos_v)
        pltpu.async_copy(mem_hbm.at[pos_v], vis_v, sem).wait()
        pltpu.sync_copy(vis_v, vis_hbm.at[b])


def _sc_gather(pos, mem2d):
    mesh = plsc.VectorSubcoreMesh(core_axis_name="c", subcore_axis_name="s")
    f = pl.kernel(
        _gather_body,
        out_type=jax.ShapeDtypeStruct((B, CP, CW), jnp.float32),
        mesh=mesh,
        scratch_types=[
            pltpu.VMEM((CP,), jnp.int32),
            pltpu.VMEM((CP, CW), jnp.float32),
            pltpu.SemaphoreType.DMA,
        ],
        compiler_params=_SC_PARAMS,
    )
    return f(pos, mem2d)


def _scatter_body(pos_hbm, vn_hbm, nm_ref, pos_v, vn_v, sem):
    wid = lax.axis_index("s") * _NC + lax.axis_index("c")
    for j in range(_BPW):
        b = wid * _BPW + j
        pltpu.sync_copy(pos_hbm.at[b], pos_v)
        pltpu.sync_copy(vn_hbm.at[b], vn_v)
        pltpu.async_copy(vn_v, nm_ref.at[pos_v], sem).wait()


def _sc_scatter(pos, vn, nm_ref):
    mesh = plsc.VectorSubcoreMesh(core_axis_name="c", subcore_axis_name="s")
    f = pl.kernel(
        _scatter_body,
        out_type=(),
        mesh=mesh,
        scratch_types=[
            pltpu.VMEM((CP,), jnp.int32),
            pltpu.VMEM((CP, CW), jnp.float32),
            pltpu.SemaphoreType.DMA,
        ],
        compiler_params=_SC_PARAMS,
    )
    return f(pos, vn, nm_ref)


def _attn_body(q_ref, vis_ref, wv_ref, g_ref, rv_ref, vn_ref):
    q = q_ref[0]
    vis = vis_ref[0]
    s = lax.dot_general(
        q, vis, (((1,), (1,)), ((), ())),
        preferred_element_type=jnp.float32)
    cols = lax.broadcasted_iota(jnp.int32, (R, CP), 1)
    s = jnp.where(cols < C, s, _NEG)
    mx = jnp.max(s, axis=1, keepdims=True)
    e = jnp.exp(s - mx)
    e = jnp.where(cols < C, e, 0.0)
    w = e / jnp.sum(e, axis=1, keepdims=True)
    rv_ref[0] = lax.dot_general(
        w, vis, (((1,), (0,)), ((), ())),
        preferred_element_type=jnp.float32)

    gv = g_ref[0, 0]
    ig = gv[0]
    wg = gv[1]
    ww = wg * (ig * jnp.mean(w, axis=0) + (1.0 - ig) / C)
    wvec = wv_ref[0, 0]
    vn = (vis * (1.0 - ww[:, None]) + ww[:, None] * wvec[None, :])
    row_lru = vn[C - 1 : C, :]
    rows = lax.broadcasted_iota(jnp.int32, (CP, CW), 0)
    vn_ref[0] = jnp.where(rows < C, vn, row_lru)


def _attn(q3, vis, wv, g):
    return pl.pallas_call(
        _attn_body,
        grid=(B,),
        in_specs=[
            pl.BlockSpec((1, R, CW), lambda b: (b, 0, 0)),
            pl.BlockSpec((1, CP, CW), lambda b: (b, 0, 0)),
            pl.BlockSpec((1, 1, CW), lambda b: (b, 0, 0)),
            pl.BlockSpec((1, 1, 16), lambda b: (b, 0, 0)),
        ],
        out_specs=[
            pl.BlockSpec((1, R, CW), lambda b: (b, 0, 0)),
            pl.BlockSpec((1, CP, CW), lambda b: (b, 0, 0)),
        ],
        out_shape=(
            jax.ShapeDtypeStruct((B, R, CW), jnp.float32),
            jax.ShapeDtypeStruct((B, CP, CW), jnp.float32),
        ),
        interpret=_INTERPRET,
    )(q3, vis, wv.reshape(B, 1, CW), g.reshape(B, 1, 16))


def kernel(x, memory, W, b):
    q, wv, g = _iface(x, W, b)
    q3 = q.reshape(B, R, CW)
    newmem, idx = _scan(q3, memory)

    idxf = idx.reshape(B, R * K)
    pos = jnp.concatenate(
        [idxf, jnp.zeros((B, CP - R * K), jnp.int32)], axis=1)
    pos = pos + (jnp.arange(B, dtype=jnp.int32) * M)[:, None]

    mem2d = memory.reshape(B * M, CW)
    vis = _sc_gather(pos, mem2d)
    rv, vn = _attn(q3, vis, wv, g)

    nm_ref = jax.new_ref(newmem.reshape(B * M, CW))
    _sc_scatter(pos, vn, nm_ref)
    new_memory = nm_ref[...].reshape(B, M, CW)
    return rv.reshape(B, R * CW), new_memory
